# baseline (speedup 1.0000x reference)
import functools

import jax
import jax.numpy as jnp
from jax import lax
from jax.experimental import pallas as pl
from jax.experimental.pallas import tpu as pltpu
from jax.experimental.pallas import tpu_sc as plsc

N = 29440
E = 471040
IN_F = 115
H = 64
G = 256
SEG = 115
NW = 32
EPW = E // NW
CH = 128
NCH = EPW // CH
NPT = N // 16
ZR = 184

_mesh = plsc.VectorSubcoreMesh(core_axis_name="c", subcore_axis_name="s")


@functools.partial(
    pl.kernel,
    out_type=jax.ShapeDtypeStruct((2 * N,), jnp.float32),
    mesh=_mesh,
    scratch_types=[
        pltpu.VMEM_SHARED((N,), jnp.float32),
        pltpu.VMEM((NCH, CH), jnp.int32),
        pltpu.VMEM((NCH, CH), jnp.float32),
        pltpu.VMEM((NPT,), jnp.float32),
    ],
)
def _k1_deg(cidx_hbm, ew_hbm, degp_hbm, deg_sp, cidx_v, ew_v, zbuf):
    cid = lax.axis_index("c")
    sid = lax.axis_index("s")
    wid = sid * 2 + cid

    pltpu.sync_copy(cidx_hbm.at[wid], cidx_v)
    pltpu.sync_copy(ew_hbm.at[wid], ew_v)

    def zf(i, _):
        zbuf[pl.ds(i * 16, 16)] = jnp.zeros((16,), jnp.float32)
        return 0
    lax.fori_loop(0, NPT // 16, zf, 0)
    pltpu.sync_copy(zbuf, deg_sp.at[pl.ds(sid * NPT, NPT)])
    plsc.subcore_barrier()

    def body(j, _):
        pltpu.sync_copy(ew_v.at[j], deg_sp.at[cidx_v.at[j]], add=True)
        return 0
    lax.fori_loop(0, NCH, body, 0)
    plsc.subcore_barrier()

    pltpu.sync_copy(deg_sp.at[pl.ds(sid * NPT, NPT)], zbuf)
    pltpu.sync_copy(zbuf, degp_hbm.at[pl.ds(cid * N + sid * NPT, NPT)])


CB = 80
NCB = EPW // CB
_HALF = NCB // 2
_NZC = NPT // CB
_TL = NPT - _NZC * CB


@functools.partial(
    pl.kernel,
    out_type=jax.ShapeDtypeStruct((2, N, H), jnp.float32),
    mesh=_mesh,
    compiler_params=pltpu.CompilerParams(use_tc_tiling_on_sc=False),
    scratch_types=[
        pltpu.VMEM_SHARED((N, H), jnp.float32),
        pltpu.VMEM((CB,), jnp.int32),
        pltpu.VMEM((CB,), jnp.int32),
        pltpu.VMEM((CB,), jnp.int32),
        pltpu.VMEM((CB,), jnp.int32),
        pltpu.VMEM((CB // 8, 128), jnp.float32),
        pltpu.VMEM((CB // 8, 128), jnp.float32),
        pltpu.VMEM((CB, H), jnp.float32),
        pltpu.VMEM((CB, H), jnp.float32),
        pltpu.SemaphoreType.DMA,
        pltpu.SemaphoreType.DMA,
        pltpu.SemaphoreType.DMA,
        pltpu.SemaphoreType.DMA,
        pltpu.SemaphoreType.DMA,
    ],
)
def _k3_msg(y_hbm, eidx_hbm, ew_hbm, accp_hbm,
            acc_sp, ridx0, ridx1, cidx0, cidx1, ew0, ew1, rows0, rows1,
            sem_i0, sem_i1, sem_g0, sem_g1, sem_s0):
    cid = lax.axis_index("c")
    sid = lax.axis_index("s")
    wid = sid * 2 + cid
    base = sid * NPT
    ridx = (ridx0, ridx1)
    cidx = (cidx0, cidx1)
    ew = (ew0, ew1)
    rows = (rows0, rows1)
    sem_i = (sem_i0, sem_i1)
    sem_g = (sem_g0, sem_g1)

    def start_idx(c, s):
        pltpu.async_copy(eidx_hbm.at[0, pl.ds(wid * EPW + c * CB, CB)],
                         ridx[s], sem_i[s])
        pltpu.async_copy(eidx_hbm.at[1, pl.ds(wid * EPW + c * CB, CB)],
                         cidx[s], sem_i[s])
        pltpu.async_copy(
            ew_hbm.at[pl.ds((wid * EPW + c * CB) // 8, CB // 8),
                      pl.ds(0, 128)], ew[s], sem_i[s])

    def wait_idx(c, s):
        pltpu.make_async_copy(eidx_hbm.at[0, pl.ds(wid * EPW + c * CB, CB)],
                              ridx[s], sem_i[s]).wait()
        pltpu.make_async_copy(eidx_hbm.at[1, pl.ds(wid * EPW + c * CB, CB)],
                              cidx[s], sem_i[s]).wait()
        pltpu.make_async_copy(
            ew_hbm.at[pl.ds((wid * EPW + c * CB) // 8, CB // 8),
                      pl.ds(0, 128)], ew[s], sem_i[s]).wait()

    def start_gather(s):
        pltpu.async_copy(y_hbm.at[ridx[s]], rows[s], sem_g[s])

    def wait_gather(s):
        pltpu.make_async_copy(y_hbm.at[ridx[s]], rows[s], sem_g[s]).wait()

    def scale(s):
        def group(g, _):
            for k in range(8):
                r = g * 8 + k
                ewb = ew[s][g, pl.ds(k * 16, 16)]
                rows[s][r, pl.ds(0, 16)] = rows[s][r, pl.ds(0, 16)] * ewb
                rows[s][r, pl.ds(16, 16)] = rows[s][r, pl.ds(16, 16)] * ewb
                rows[s][r, pl.ds(32, 16)] = rows[s][r, pl.ds(32, 16)] * ewb
                rows[s][r, pl.ds(48, 16)] = rows[s][r, pl.ds(48, 16)] * ewb
            return 0
        lax.fori_loop(0, CB // 8, group, 0)

    def zf(i, _):
        rows0[i, pl.ds(0, 16)] = jnp.zeros((16,), jnp.float32)
        rows0[i, pl.ds(16, 16)] = jnp.zeros((16,), jnp.float32)
        rows0[i, pl.ds(32, 16)] = jnp.zeros((16,), jnp.float32)
        rows0[i, pl.ds(48, 16)] = jnp.zeros((16,), jnp.float32)
        return 0
    lax.fori_loop(0, CB, zf, 0)
    def zc(t, _):
        pltpu.sync_copy(rows0, acc_sp.at[pl.ds(base + t * CB, CB)])
        return 0
    lax.fori_loop(0, _NZC, zc, 0)
    if _TL:
        pltpu.sync_copy(rows0.at[pl.ds(0, _TL)],
                        acc_sp.at[pl.ds(base + _NZC * CB, _TL)])
    plsc.subcore_barrier()

    start_idx(0, 0)
    start_idx(1, 1)
    wait_idx(0, 0)
    start_gather(0)

    def iter_t(t, _):
        a = 2 * t
        wait_idx(a + 1, 1)
        start_gather(1)
        wait_gather(0)
        @pl.when(t < _HALF - 1)
        def _():
            pltpu.async_copy(
                eidx_hbm.at[0, pl.ds(wid * EPW + (a + 2) * CB, CB)],
                ridx0, sem_i0)
        scale(0)
        pltpu.async_copy(rows0, acc_sp.at[cidx0], sem_s0, add=True)
        wait_gather(1)
        @pl.when(t < _HALF - 1)
        def _():
            pltpu.async_copy(
                eidx_hbm.at[0, pl.ds(wid * EPW + (a + 3) * CB, CB)],
                ridx1, sem_i1)
        scale(1)
        pltpu.make_async_copy(rows0, acc_sp.at[cidx0], sem_s0).wait()
        @pl.when(t < _HALF - 1)
        def _():
            pltpu.async_copy(
                eidx_hbm.at[1, pl.ds(wid * EPW + (a + 2) * CB, CB)],
                cidx0, sem_i0)
            pltpu.async_copy(
                ew_hbm.at[pl.ds((wid * EPW + (a + 2) * CB) // 8, CB // 8),
                          pl.ds(0, 128)], ew0, sem_i0)
            wait_idx(a + 2, 0)
            start_gather(0)
        pltpu.sync_copy(rows1, acc_sp.at[cidx1], add=True)
        @pl.when(t < _HALF - 1)
        def _():
            pltpu.async_copy(
                eidx_hbm.at[1, pl.ds(wid * EPW + (a + 3) * CB, CB)],
                cidx1, sem_i1)
            pltpu.async_copy(
                ew_hbm.at[pl.ds((wid * EPW + (a + 3) * CB) // 8, CB // 8),
                          pl.ds(0, 128)], ew1, sem_i1)
        return 0
    lax.fori_loop(0, _HALF, iter_t, 0)
    plsc.subcore_barrier()

    def out_chunk(t, _):
        pltpu.sync_copy(acc_sp.at[pl.ds(base + t * CB, CB)], rows0)
        pltpu.sync_copy(rows0, accp_hbm.at[cid, pl.ds(base + t * CB, CB)])
        return 0
    lax.fori_loop(0, _NZC, out_chunk, 0)
    if _TL:
        pltpu.sync_copy(acc_sp.at[pl.ds(base + _NZC * CB, _TL)],
                        rows0.at[pl.ds(0, _TL)])
        pltpu.sync_copy(rows0.at[pl.ds(0, _TL)],
                        accp_hbm.at[cid, pl.ds(base + _NZC * CB, _TL)])


def _k2_body(x_ref, w_ref, degp_ref, ew8_ref, y_ref, dis_ref, ewrep_ref):
    deg = degp_ref[0, 0, :] + degp_ref[0, 1, :] + 1.0
    dis = lax.rsqrt(deg)
    xw = jnp.dot(x_ref[...], w_ref[...], preferred_element_type=jnp.float32)
    y_ref[...] = xw * dis[:, None]
    dis_ref[...] = dis[:, None]
    jio = lax.broadcasted_iota(jnp.int32, (8, 128), 0)
    lio = lax.broadcasted_iota(jnp.int32, (8, 128), 1) // 16
    rep = (jio == lio).astype(jnp.float32)
    ewrep_ref[...] = jnp.dot(ew8_ref[...], rep,
                             preferred_element_type=jnp.float32)


def _k2_call(x, w, degp, ew8):
    bn = N // 16
    be = E // 8 // 16
    degpt = degp.reshape(2, 16, bn).transpose(1, 0, 2)
    return pl.pallas_call(
        _k2_body,
        grid=(16,),
        in_specs=[
            pl.BlockSpec((bn, IN_F), lambda i: (i, 0)),
            pl.BlockSpec((IN_F, H), lambda i: (0, 0)),
            pl.BlockSpec((1, 2, bn), lambda i: (i, 0, 0)),
            pl.BlockSpec((be, 8), lambda i: (i, 0)),
        ],
        out_specs=[
            pl.BlockSpec((bn, H), lambda i: (i, 0)),
            pl.BlockSpec((bn, 1), lambda i: (i, 0)),
            pl.BlockSpec((be, 128), lambda i: (i, 0)),
        ],
        out_shape=(jax.ShapeDtypeStruct((N, H), jnp.float32),
                   jax.ShapeDtypeStruct((N, 1), jnp.float32),
                   jax.ShapeDtypeStruct((E // 8, 128), jnp.float32)),
    )(x, w, degpt, ew8)


def _k0_body(w1_ref, w2r_ref, b1_ref, b2_ref, v_ref, c_ref):
    w2r = w2r_ref[...]
    v_ref[...] = jnp.sum(w1_ref[...] * w2r, axis=1, keepdims=True)
    c_ref[...] = (jnp.sum(b1_ref[...] * w2r, axis=1, keepdims=True)
                  + b2_ref[...])


def _k0_call(w1, w2, b1, b2):
    return pl.pallas_call(
        _k0_body,
        out_shape=(jax.ShapeDtypeStruct((SEG * H, 1), jnp.float32),
                   jax.ShapeDtypeStruct((1, 1), jnp.float32)),
    )(w1, w2.reshape(1, -1), b1.reshape(1, -1), b2.reshape(1, 1))


def _k4_body(accp_ref, y_ref, dis_ref, b_ref, vt_ref, c_ref, z_ref):
    gb = z_ref.shape[0]
    rows = gb * SEG
    h = accp_ref[0] + accp_ref[1] + y_ref[...]
    h = jax.nn.relu(h * dis_ref[...] + b_ref[...])
    p = h * vt_ref[...]
    rsum = jnp.sum(p, axis=1, keepdims=True)
    gidx = lax.broadcasted_iota(jnp.int32, (gb, rows), 0)
    ridx = lax.broadcasted_iota(jnp.int32, (gb, rows), 1) // SEG
    sel = (gidx == ridx).astype(jnp.float32)
    z_ref[...] = (jnp.dot(sel, rsum, preferred_element_type=jnp.float32)
                  + c_ref[...])


def _k4_call(accp, y, dis, b, vtile, c):
    gb = 8
    rows = gb * SEG
    grid = G // gb
    return pl.pallas_call(
        _k4_body,
        grid=(grid,),
        in_specs=[
            pl.BlockSpec((2, rows, H), lambda i: (0, i, 0)),
            pl.BlockSpec((rows, H), lambda i: (i, 0)),
            pl.BlockSpec((rows, 1), lambda i: (i, 0)),
            pl.BlockSpec((1, H), lambda i: (0, 0)),
            pl.BlockSpec((rows, H), lambda i: (0, 0)),
            pl.BlockSpec((1, 1), lambda i: (0, 0)),
        ],
        out_specs=pl.BlockSpec((gb, 1), lambda i: (i, 0)),
        out_shape=jax.ShapeDtypeStruct((G, 1), jnp.float32),
    )(accp, y, dis, b, vtile, c)


def kernel(x, edge_index, edge_weight, batch, device, W, b, W1, b1, W2, b2):
    col = edge_index[1]
    cidx3 = col.reshape(NW, NCH, CH)
    ew3 = edge_weight.reshape(NW, NCH, CH)

    degp = _k1_deg(cidx3, ew3)
    y, dis, ewrep = _k2_call(x, W, degp, edge_weight.reshape(E // 8, 8))
    accp = _k3_msg(y, edge_index, ewrep)
    v, c = _k0_call(W1, W2, b1, b2)
    vtile = jnp.tile(v.reshape(SEG, H), (8, 1))
    z = _k4_call(accp, y, dis, b.reshape(1, H), vtile, c)
    return z


# speedup vs baseline: 1.1159x; 1.1159x over previous
import functools

import jax
import jax.numpy as jnp
from jax import lax
from jax.experimental import pallas as pl
from jax.experimental.pallas import tpu as pltpu
from jax.experimental.pallas import tpu_sc as plsc

N = 29440
E = 471040
IN_F = 115
H = 64
G = 256
SEG = 115
NW = 32
EPW = E // NW
CH = 128
NCH = EPW // CH
NPT = N // 16
ZR = 184

_mesh = plsc.VectorSubcoreMesh(core_axis_name="c", subcore_axis_name="s")


@functools.partial(
    pl.kernel,
    out_type=jax.ShapeDtypeStruct((2 * N,), jnp.float32),
    mesh=_mesh,
    scratch_types=[
        pltpu.VMEM_SHARED((N,), jnp.float32),
        pltpu.VMEM((NCH, CH), jnp.int32),
        pltpu.VMEM((NCH, CH), jnp.float32),
        pltpu.VMEM((NPT,), jnp.float32),
    ],
)
def _k1_deg(cidx_hbm, ew_hbm, degp_hbm, deg_sp, cidx_v, ew_v, zbuf):
    cid = lax.axis_index("c")
    sid = lax.axis_index("s")
    wid = sid * 2 + cid

    pltpu.sync_copy(cidx_hbm.at[wid], cidx_v)
    pltpu.sync_copy(ew_hbm.at[wid], ew_v)

    def zf(i, _):
        zbuf[pl.ds(i * 16, 16)] = jnp.zeros((16,), jnp.float32)
        return 0
    lax.fori_loop(0, NPT // 16, zf, 0)
    pltpu.sync_copy(zbuf, deg_sp.at[pl.ds(sid * NPT, NPT)])
    plsc.subcore_barrier()

    def body(j, _):
        pltpu.sync_copy(ew_v.at[j], deg_sp.at[cidx_v.at[j]], add=True)
        return 0
    lax.fori_loop(0, NCH, body, 0)
    plsc.subcore_barrier()

    pltpu.sync_copy(deg_sp.at[pl.ds(sid * NPT, NPT)], zbuf)
    pltpu.sync_copy(zbuf, degp_hbm.at[pl.ds(cid * N + sid * NPT, NPT)])


CB = 80
NCB = EPW // CB
_HALF = NCB // 2
_NZC = NPT // CB
_TL = NPT - _NZC * CB


@functools.partial(
    pl.kernel,
    out_type=jax.ShapeDtypeStruct((2, N, H), jnp.float32),
    mesh=_mesh,
    compiler_params=pltpu.CompilerParams(use_tc_tiling_on_sc=False),
    scratch_types=[
        pltpu.VMEM_SHARED((N, H), jnp.float32),
        pltpu.VMEM((CB,), jnp.int32),
        pltpu.VMEM((CB,), jnp.int32),
        pltpu.VMEM((CB,), jnp.int32),
        pltpu.VMEM((CB,), jnp.int32),
        pltpu.VMEM((CB // 8, 128), jnp.float32),
        pltpu.VMEM((CB // 8, 128), jnp.float32),
        pltpu.VMEM((CB, H), jnp.float32),
        pltpu.VMEM((CB, H), jnp.float32),
        pltpu.SemaphoreType.DMA,
        pltpu.SemaphoreType.DMA,
        pltpu.SemaphoreType.DMA,
        pltpu.SemaphoreType.DMA,
    ],
)
def _k3_msg(y_hbm, eidx_hbm, ew_hbm, accp_hbm,
            acc_sp, ridx0, ridx1, cidx0, cidx1, ew0, ew1, rows0, rows1,
            sem_i0, sem_i1, sem_g0, sem_g1):
    cid = lax.axis_index("c")
    sid = lax.axis_index("s")
    wid = sid * 2 + cid
    base = sid * NPT
    ridx = (ridx0, ridx1)
    cidx = (cidx0, cidx1)
    ew = (ew0, ew1)
    rows = (rows0, rows1)
    sem_i = (sem_i0, sem_i1)
    sem_g = (sem_g0, sem_g1)

    def start_idx(c, s):
        pltpu.async_copy(eidx_hbm.at[0, pl.ds(wid * EPW + c * CB, CB)],
                         ridx[s], sem_i[s])
        pltpu.async_copy(eidx_hbm.at[1, pl.ds(wid * EPW + c * CB, CB)],
                         cidx[s], sem_i[s])
        pltpu.async_copy(
            ew_hbm.at[pl.ds((wid * EPW + c * CB) // 8, CB // 8),
                      pl.ds(0, 128)], ew[s], sem_i[s])

    def wait_idx(c, s):
        pltpu.make_async_copy(eidx_hbm.at[0, pl.ds(wid * EPW + c * CB, CB)],
                              ridx[s], sem_i[s]).wait()
        pltpu.make_async_copy(eidx_hbm.at[1, pl.ds(wid * EPW + c * CB, CB)],
                              cidx[s], sem_i[s]).wait()
        pltpu.make_async_copy(
            ew_hbm.at[pl.ds((wid * EPW + c * CB) // 8, CB // 8),
                      pl.ds(0, 128)], ew[s], sem_i[s]).wait()

    def start_gather(s):
        pltpu.async_copy(y_hbm.at[ridx[s]], rows[s], sem_g[s])

    def wait_gather(s):
        pltpu.make_async_copy(y_hbm.at[ridx[s]], rows[s], sem_g[s]).wait()

    def scale(s):
        def group(g, _):
            for k in range(8):
                r = g * 8 + k
                ewb = ew[s][g, pl.ds(k * 16, 16)]
                rows[s][r, pl.ds(0, 16)] = rows[s][r, pl.ds(0, 16)] * ewb
                rows[s][r, pl.ds(16, 16)] = rows[s][r, pl.ds(16, 16)] * ewb
                rows[s][r, pl.ds(32, 16)] = rows[s][r, pl.ds(32, 16)] * ewb
                rows[s][r, pl.ds(48, 16)] = rows[s][r, pl.ds(48, 16)] * ewb
            return 0
        lax.fori_loop(0, CB // 8, group, 0)

    def zf(i, _):
        rows0[i, pl.ds(0, 16)] = jnp.zeros((16,), jnp.float32)
        rows0[i, pl.ds(16, 16)] = jnp.zeros((16,), jnp.float32)
        rows0[i, pl.ds(32, 16)] = jnp.zeros((16,), jnp.float32)
        rows0[i, pl.ds(48, 16)] = jnp.zeros((16,), jnp.float32)
        return 0
    lax.fori_loop(0, CB, zf, 0)
    def zc(t, _):
        pltpu.sync_copy(rows0, acc_sp.at[pl.ds(base + t * CB, CB)])
        return 0
    lax.fori_loop(0, _NZC, zc, 0)
    if _TL:
        pltpu.sync_copy(rows0.at[pl.ds(0, _TL)],
                        acc_sp.at[pl.ds(base + _NZC * CB, _TL)])
    plsc.subcore_barrier()

    start_idx(0, 0)
    start_idx(1, 1)
    wait_idx(0, 0)
    start_gather(0)

    def iter_t(t, _):
        a = 2 * t
        wait_idx(a + 1, 1)
        start_gather(1)
        wait_gather(0)
        @pl.when(t < _HALF - 1)
        def _():
            pltpu.async_copy(
                eidx_hbm.at[0, pl.ds(wid * EPW + (a + 2) * CB, CB)],
                ridx0, sem_i0)
        scale(0)
        pltpu.sync_copy(rows0, acc_sp.at[cidx0], add=True)
        @pl.when(t < _HALF - 1)
        def _():
            pltpu.async_copy(
                eidx_hbm.at[1, pl.ds(wid * EPW + (a + 2) * CB, CB)],
                cidx0, sem_i0)
            pltpu.async_copy(
                ew_hbm.at[pl.ds((wid * EPW + (a + 2) * CB) // 8, CB // 8),
                          pl.ds(0, 128)], ew0, sem_i0)
        wait_gather(1)
        @pl.when(t < _HALF - 1)
        def _():
            pltpu.async_copy(
                eidx_hbm.at[0, pl.ds(wid * EPW + (a + 3) * CB, CB)],
                ridx1, sem_i1)
        scale(1)
        @pl.when(t < _HALF - 1)
        def _():
            wait_idx(a + 2, 0)
            start_gather(0)
        pltpu.sync_copy(rows1, acc_sp.at[cidx1], add=True)
        @pl.when(t < _HALF - 1)
        def _():
            pltpu.async_copy(
                eidx_hbm.at[1, pl.ds(wid * EPW + (a + 3) * CB, CB)],
                cidx1, sem_i1)
            pltpu.async_copy(
                ew_hbm.at[pl.ds((wid * EPW + (a + 3) * CB) // 8, CB // 8),
                          pl.ds(0, 128)], ew1, sem_i1)
        return 0
    lax.fori_loop(0, _HALF, iter_t, 0)
    plsc.subcore_barrier()

    def out_chunk(t, _):
        pltpu.sync_copy(acc_sp.at[pl.ds(base + t * CB, CB)], rows0)
        pltpu.sync_copy(rows0, accp_hbm.at[cid, pl.ds(base + t * CB, CB)])
        return 0
    lax.fori_loop(0, _NZC, out_chunk, 0)
    if _TL:
        pltpu.sync_copy(acc_sp.at[pl.ds(base + _NZC * CB, _TL)],
                        rows0.at[pl.ds(0, _TL)])
        pltpu.sync_copy(rows0.at[pl.ds(0, _TL)],
                        accp_hbm.at[cid, pl.ds(base + _NZC * CB, _TL)])


def _k2_body(x_ref, w_ref, degp_ref, ew8_ref, y_ref, dis_ref, ewrep_ref):
    deg = degp_ref[0, 0, :] + degp_ref[0, 1, :] + 1.0
    dis = lax.rsqrt(deg)
    xw = jnp.dot(x_ref[...], w_ref[...], preferred_element_type=jnp.float32)
    y_ref[...] = xw * dis[:, None]
    dis_ref[...] = dis[:, None]
    jio = lax.broadcasted_iota(jnp.int32, (8, 128), 0)
    lio = lax.broadcasted_iota(jnp.int32, (8, 128), 1) // 16
    rep = (jio == lio).astype(jnp.float32)
    ewrep_ref[...] = jnp.dot(ew8_ref[...], rep,
                             preferred_element_type=jnp.float32)


def _k2_call(x, w, degp, ew8):
    bn = N // 16
    be = E // 8 // 16
    degpt = degp.reshape(2, 16, bn).transpose(1, 0, 2)
    return pl.pallas_call(
        _k2_body,
        grid=(16,),
        in_specs=[
            pl.BlockSpec((bn, IN_F), lambda i: (i, 0)),
            pl.BlockSpec((IN_F, H), lambda i: (0, 0)),
            pl.BlockSpec((1, 2, bn), lambda i: (i, 0, 0)),
            pl.BlockSpec((be, 8), lambda i: (i, 0)),
        ],
        out_specs=[
            pl.BlockSpec((bn, H), lambda i: (i, 0)),
            pl.BlockSpec((bn, 1), lambda i: (i, 0)),
            pl.BlockSpec((be, 128), lambda i: (i, 0)),
        ],
        out_shape=(jax.ShapeDtypeStruct((N, H), jnp.float32),
                   jax.ShapeDtypeStruct((N, 1), jnp.float32),
                   jax.ShapeDtypeStruct((E // 8, 128), jnp.float32)),
    )(x, w, degpt, ew8)


def _k0_body(w1_ref, w2r_ref, b1_ref, b2_ref, v_ref, c_ref):
    w2r = w2r_ref[...]
    v_ref[...] = jnp.sum(w1_ref[...] * w2r, axis=1, keepdims=True)
    c_ref[...] = (jnp.sum(b1_ref[...] * w2r, axis=1, keepdims=True)
                  + b2_ref[...])


def _k0_call(w1, w2, b1, b2):
    return pl.pallas_call(
        _k0_body,
        out_shape=(jax.ShapeDtypeStruct((SEG * H, 1), jnp.float32),
                   jax.ShapeDtypeStruct((1, 1), jnp.float32)),
    )(w1, w2.reshape(1, -1), b1.reshape(1, -1), b2.reshape(1, 1))


def _k4_body(accp_ref, y_ref, dis_ref, b_ref, vt_ref, c_ref, z_ref):
    gb = z_ref.shape[0]
    rows = gb * SEG
    h = accp_ref[0] + accp_ref[1] + y_ref[...]
    h = jax.nn.relu(h * dis_ref[...] + b_ref[...])
    p = h * vt_ref[...]
    rsum = jnp.sum(p, axis=1, keepdims=True)
    gidx = lax.broadcasted_iota(jnp.int32, (gb, rows), 0)
    ridx = lax.broadcasted_iota(jnp.int32, (gb, rows), 1) // SEG
    sel = (gidx == ridx).astype(jnp.float32)
    z_ref[...] = (jnp.dot(sel, rsum, preferred_element_type=jnp.float32)
                  + c_ref[...])


def _k4_call(accp, y, dis, b, vtile, c):
    gb = 8
    rows = gb * SEG
    grid = G // gb
    return pl.pallas_call(
        _k4_body,
        grid=(grid,),
        in_specs=[
            pl.BlockSpec((2, rows, H), lambda i: (0, i, 0)),
            pl.BlockSpec((rows, H), lambda i: (i, 0)),
            pl.BlockSpec((rows, 1), lambda i: (i, 0)),
            pl.BlockSpec((1, H), lambda i: (0, 0)),
            pl.BlockSpec((rows, H), lambda i: (0, 0)),
            pl.BlockSpec((1, 1), lambda i: (0, 0)),
        ],
        out_specs=pl.BlockSpec((gb, 1), lambda i: (i, 0)),
        out_shape=jax.ShapeDtypeStruct((G, 1), jnp.float32),
    )(accp, y, dis, b, vtile, c)


def kernel(x, edge_index, edge_weight, batch, device, W, b, W1, b1, W2, b2):
    col = edge_index[1]
    cidx3 = col.reshape(NW, NCH, CH)
    ew3 = edge_weight.reshape(NW, NCH, CH)

    degp = _k1_deg(cidx3, ew3)
    y, dis, ewrep = _k2_call(x, W, degp, edge_weight.reshape(E // 8, 8))
    accp = _k3_msg(y, edge_index, ewrep)
    v, c = _k0_call(W1, W2, b1, b2)
    vtile = jnp.tile(v.reshape(SEG, H), (8, 1))
    z = _k4_call(accp, y, dis, b.reshape(1, H), vtile, c)
    return z
